# baseline (speedup 1.0000x reference)
import functools

import jax
import jax.numpy as jnp
from jax import lax
from jax.experimental import pallas as pl
from jax.experimental.pallas import tpu as pltpu
from jax.experimental.pallas import tpu_sc as plsc

N_EX = 1000
N_KC = 100
D = 128
NHEADS = 4
ALPHA = 0.2

_F32 = jnp.float32


def _dense_body(xe_ref, xk_ref, adjek_ref, adjee_ref, adjkk_ref, watt_ref,
                asrc_ref, adst_ref, wee_ref, wkk_ref,
                e_out, ce_out, ck_out):
    f32 = _F32
    xe = xe_ref[...]
    xk = xk_ref[...]

    ce_out[...] = jnp.maximum(
        jnp.dot(adjee_ref[...], jnp.dot(xe, wee_ref[...],
                                        preferred_element_type=f32),
                preferred_element_type=f32), 0.0)
    ck_out[...] = jnp.maximum(
        jnp.dot(adjkk_ref[...], jnp.dot(xk, wkk_ref[...],
                                        preferred_element_type=f32),
                preferred_element_type=f32), 0.0)

    adj = adjek_ref[...]
    ones_col = jnp.ones((N_EX, 1), f32)
    row_i = lax.broadcasted_iota(jnp.int32, (D, D), 0)
    col_i = lax.broadcasted_iota(jnp.int32, (D, D), 1)

    acc = jnp.zeros((N_EX, D), f32)
    for h in range(NHEADS):
        w = watt_ref[h]
        whe = jnp.dot(xe, w, preferred_element_type=f32)
        whk = jnp.dot(xk, w, preferred_element_type=f32)
        u = jnp.dot(whe, asrc_ref[h], preferred_element_type=f32)
        v = jnp.dot(whk, adst_ref[h], preferred_element_type=f32)
        vb = lax.dot_general(ones_col, v, (((1,), (1,)), ((), ())),
                             preferred_element_type=f32)
        e = u + vb
        e = jnp.where(e > 0, e, ALPHA * e)
        e = jnp.where(adj > 0, e, -9e15)
        m = jnp.max(e, axis=1, keepdims=True)
        p = jnp.exp(e - m)
        att = p / jnp.sum(p, axis=1, keepdims=True)
        head = jnp.dot(att, whk, preferred_element_type=f32)
        head = jnp.where(head > 0, head, jnp.exp(head) - 1.0)
        r_h = jnp.where(col_i == 32 * h + row_i // 4, 0.25, 0.0).astype(f32)
        acc = acc + jnp.dot(head, r_h, preferred_element_type=f32)
    e_out[...] = acc


_dense_call = pl.pallas_call(
    _dense_body,
    out_shape=[
        jax.ShapeDtypeStruct((N_EX, D), _F32),
        jax.ShapeDtypeStruct((N_EX, D), _F32),
        jax.ShapeDtypeStruct((N_KC, D), _F32),
    ],
)


_NW = 32
_SLOTS = 4


def _pipelined_lookup(tbl, out, idx_v, bufs, gsems, wsems, ibase, obase, nch,
                      chunk):
    def body(i, carry):
        cps = []
        for k in range(_SLOTS):
            j = i * _SLOTS + k

            @pl.when(i > 0)
            def _():
                pltpu.make_async_copy(
                    bufs.at[k], out.at[pl.ds(obase, chunk)], wsems[k]).wait()

            idx_chunk = idx_v.at[pl.ds(ibase + j * chunk, chunk)]
            cps.append(pltpu.async_copy(tbl.at[idx_chunk], bufs.at[k],
                                        gsems[k]))
        for k in range(_SLOTS):
            j = i * _SLOTS + k
            cps[k].wait()
            pltpu.async_copy(bufs.at[k],
                             out.at[pl.ds(obase + j * chunk, chunk)],
                             wsems[k])
        return carry

    lax.fori_loop(0, nch // _SLOTS, body, 0)
    for k in range(_SLOTS):
        pltpu.make_async_copy(
            bufs.at[k], out.at[pl.ds(obase, chunk)], wsems[k]).wait()


def _sc_gather_body(rows_w, te, idxe, idxr2, out1, out2,
                    idx_v, bufs1, bufs2, *sems):
    gsems, wsems = sems[:_SLOTS], sems[_SLOTS:]
    sid = lax.axis_index("s")
    wid = lax.axis_index("c") * 16 + sid
    obase = wid * rows_w

    pltpu.sync_copy(idxe.at[pl.ds(obase, rows_w)], idx_v.at[pl.ds(0, rows_w)])
    _pipelined_lookup(te, out1, idx_v, bufs1, gsems, wsems,
                      0, obase, rows_w // 64, 64)
    pltpu.sync_copy(idxr2.at[pl.ds(2 * obase, 2 * rows_w)], idx_v)
    _pipelined_lookup(te, out2, idx_v, bufs2, gsems, wsems,
                      0, 2 * obase, 2 * rows_w // 128, 128)


@functools.lru_cache(maxsize=None)
def _make_sc_gather(n_rows):
    rows_w = n_rows // _NW
    mesh = plsc.VectorSubcoreMesh(core_axis_name="c", subcore_axis_name="s")
    return pl.kernel(
        functools.partial(_sc_gather_body, rows_w),
        out_type=[
            jax.ShapeDtypeStruct((n_rows, D), _F32),
            jax.ShapeDtypeStruct((2 * n_rows, D), _F32),
        ],
        mesh=mesh,
        scratch_types=[
            pltpu.VMEM((2 * rows_w,), jnp.int32),
            pltpu.VMEM((_SLOTS, 64, D), _F32),
            pltpu.VMEM((_SLOTS, 128, D), _F32),
        ] + [pltpu.SemaphoreType.DMA] * (2 * _SLOTS),
    )


def kernel(exercise_node_embedding, kc_node_mebedding, adj_exercise_kc,
           adj_EE_view, adj_KK_view, exercise_data, exercise_respond_data,
           seqlen, W_att, a_att, W_EE, W_KK):
    b, s = exercise_data.shape
    n_rows = b * s

    asrc = a_att[:, :D]
    adst = a_att[:, D:]

    ex_emb, contrastive_e, contrastive_k = _dense_call(
        exercise_node_embedding, kc_node_mebedding, adj_exercise_kc,
        adj_EE_view, adj_KK_view, W_att, asrc, adst, W_EE, W_KK)

    table_e = jnp.concatenate([jnp.zeros((1, D), _F32), ex_emb], axis=0)

    idxe = exercise_data.T.astype(jnp.int32).reshape(n_rows)
    q = exercise_respond_data.T.astype(jnp.int32).reshape(n_rows)
    idxr2 = jnp.stack(
        [jnp.maximum(q - N_EX, 0), jnp.where(q <= N_EX, q, 0)],
        axis=-1).reshape(2 * n_rows)

    out1, out2 = _make_sc_gather(n_rows)(table_e, idxe, idxr2)

    return (out2.reshape(s, b, 2 * D),
            out1.reshape(s, b, D),
            ex_emb, contrastive_e, contrastive_k)


# speedup vs baseline: 26.8839x; 26.8839x over previous
import functools

import jax
import jax.numpy as jnp
from jax import lax
from jax.experimental import pallas as pl
from jax.experimental.pallas import tpu as pltpu
from jax.experimental.pallas import tpu_sc as plsc

N_EX = 1000
N_KC = 100
D = 128
NHEADS = 4
ALPHA = 0.2

_F32 = jnp.float32


def _dense_body(xe_ref, xk_ref, adjek_ref, adjee_ref, adjkk_ref, watt_ref,
                asrc_ref, adst_ref, wee_ref, wkk_ref,
                e_out, ce_out, ck_out):
    f32 = _F32
    xe = xe_ref[...]
    xk = xk_ref[...]

    ce_out[...] = jnp.maximum(
        jnp.dot(adjee_ref[...], jnp.dot(xe, wee_ref[...],
                                        preferred_element_type=f32),
                preferred_element_type=f32), 0.0)
    ck_out[...] = jnp.maximum(
        jnp.dot(adjkk_ref[...], jnp.dot(xk, wkk_ref[...],
                                        preferred_element_type=f32),
                preferred_element_type=f32), 0.0)

    adj = adjek_ref[...]
    ones_col = jnp.ones((N_EX, 1), f32)
    row_i = lax.broadcasted_iota(jnp.int32, (D, D), 0)
    col_i = lax.broadcasted_iota(jnp.int32, (D, D), 1)

    acc = jnp.zeros((N_EX, D), f32)
    for h in range(NHEADS):
        w = watt_ref[h]
        whe = jnp.dot(xe, w, preferred_element_type=f32)
        whk = jnp.dot(xk, w, preferred_element_type=f32)
        u = jnp.dot(whe, asrc_ref[h], preferred_element_type=f32)
        v = jnp.dot(whk, adst_ref[h], preferred_element_type=f32)
        vb = lax.dot_general(ones_col, v, (((1,), (1,)), ((), ())),
                             preferred_element_type=f32)
        e = u + vb
        e = jnp.where(e > 0, e, ALPHA * e)
        e = jnp.where(adj > 0, e, -9e15)
        m = jnp.max(e, axis=1, keepdims=True)
        p = jnp.exp(e - m)
        att = p / jnp.sum(p, axis=1, keepdims=True)
        head = jnp.dot(att, whk, preferred_element_type=f32)
        head = jnp.where(head > 0, head, jnp.exp(head) - 1.0)
        r_h = jnp.where(col_i == 32 * h + row_i // 4, 0.25, 0.0).astype(f32)
        acc = acc + jnp.dot(head, r_h, preferred_element_type=f32)
    e_out[...] = acc


_dense_call = pl.pallas_call(
    _dense_body,
    out_shape=[
        jax.ShapeDtypeStruct((N_EX, D), _F32),
        jax.ShapeDtypeStruct((N_EX, D), _F32),
        jax.ShapeDtypeStruct((N_KC, D), _F32),
    ],
)


_NW = 32
_SLOTS = 4


def _pipelined_lookup(tbl, out, idx_v, bufs, gsems, wsems, ibase, obase, nch,
                      chunk):
    def body(i, carry):
        cps = []
        for k in range(_SLOTS):
            j = i * _SLOTS + k

            @pl.when(i > 0)
            def _():
                pltpu.make_async_copy(
                    bufs.at[k], out.at[pl.ds(obase, chunk)], wsems[k]).wait()

            idx_chunk = idx_v.at[pl.ds(ibase + j * chunk, chunk)]
            cps.append(pltpu.async_copy(tbl.at[idx_chunk], bufs.at[k],
                                        gsems[k]))
        for k in range(_SLOTS):
            j = i * _SLOTS + k
            cps[k].wait()
            pltpu.async_copy(bufs.at[k],
                             out.at[pl.ds(obase + j * chunk, chunk)],
                             wsems[k])
        return carry

    lax.fori_loop(0, nch // _SLOTS, body, 0)
    for k in range(_SLOTS):
        pltpu.make_async_copy(
            bufs.at[k], out.at[pl.ds(obase, chunk)], wsems[k]).wait()


def _sc_gather_body(rows_w, te, tr, idxe, idxr, out1, out2,
                    idx_v, bufs1, bufs2, *sems):
    gsems, wsems = sems[:_SLOTS], sems[_SLOTS:]
    sid = lax.axis_index("s")
    wid = lax.axis_index("c") * 16 + sid
    obase = wid * rows_w

    pltpu.sync_copy(idxe.at[pl.ds(obase, rows_w)], idx_v)
    _pipelined_lookup(te, out1, idx_v, bufs1, gsems, wsems,
                      0, obase, rows_w // 64, 64)
    pltpu.sync_copy(idxr.at[pl.ds(obase, rows_w)], idx_v)
    _pipelined_lookup(tr, out2, idx_v, bufs2, gsems, wsems,
                      0, obase, rows_w // 64, 64)


@functools.lru_cache(maxsize=None)
def _make_sc_gather(n_rows):
    rows_w = n_rows // _NW
    mesh = plsc.VectorSubcoreMesh(core_axis_name="c", subcore_axis_name="s")
    return pl.kernel(
        functools.partial(_sc_gather_body, rows_w),
        out_type=[
            jax.ShapeDtypeStruct((n_rows, D), _F32),
            jax.ShapeDtypeStruct((n_rows, 2 * D), _F32),
        ],
        mesh=mesh,
        scratch_types=[
            pltpu.VMEM((rows_w,), jnp.int32),
            pltpu.VMEM((_SLOTS, 64, D), _F32),
            pltpu.VMEM((_SLOTS, 64, 2 * D), _F32),
        ] + [pltpu.SemaphoreType.DMA] * (2 * _SLOTS),
    )


def kernel(exercise_node_embedding, kc_node_mebedding, adj_exercise_kc,
           adj_EE_view, adj_KK_view, exercise_data, exercise_respond_data,
           seqlen, W_att, a_att, W_EE, W_KK):
    b, s = exercise_data.shape
    n_rows = b * s

    asrc = a_att[:, :D]
    adst = a_att[:, D:]

    ex_emb, contrastive_e, contrastive_k = _dense_call(
        exercise_node_embedding, kc_node_mebedding, adj_exercise_kc,
        adj_EE_view, adj_KK_view, W_att, asrc, adst, W_EE, W_KK)

    table_e = jnp.concatenate([jnp.zeros((1, D), _F32), ex_emb], axis=0)
    z = jnp.zeros_like(ex_emb)
    table_r = jnp.concatenate([
        jnp.zeros((1, 2 * D), _F32),
        jnp.concatenate([z, ex_emb], axis=1),
        jnp.concatenate([ex_emb, z], axis=1),
    ], axis=0)

    idxe = exercise_data.T.astype(jnp.int32).reshape(n_rows)
    idxr = exercise_respond_data.T.astype(jnp.int32).reshape(n_rows)

    out1, out2 = _make_sc_gather(n_rows)(table_e, table_r, idxe, idxr)

    return (out2.reshape(s, b, 2 * D),
            out1.reshape(s, b, D),
            ex_emb, contrastive_e, contrastive_k)


# speedup vs baseline: 30.0001x; 1.1159x over previous
import functools

import jax
import jax.numpy as jnp
from jax import lax
from jax.experimental import pallas as pl
from jax.experimental.pallas import tpu as pltpu
from jax.experimental.pallas import tpu_sc as plsc

N_EX = 1000
N_KC = 100
D = 128
NHEADS = 4
ALPHA = 0.2

_F32 = jnp.float32


def _dense_body(xe_ref, xk_ref, adjek_ref, adjee_ref, adjkk_ref, watt_ref,
                asrc_ref, adst_ref, wee_ref, wkk_ref,
                e_out, ce_out, ck_out):
    f32 = _F32
    xe = xe_ref[...]
    xk = xk_ref[...]

    ce_out[...] = jnp.maximum(
        jnp.dot(adjee_ref[...], jnp.dot(xe, wee_ref[...],
                                        preferred_element_type=f32),
                preferred_element_type=f32), 0.0)
    ck_out[...] = jnp.maximum(
        jnp.dot(adjkk_ref[...], jnp.dot(xk, wkk_ref[...],
                                        preferred_element_type=f32),
                preferred_element_type=f32), 0.0)

    adj = adjek_ref[...]
    ones_col = jnp.ones((N_EX, 1), f32)
    row_i = lax.broadcasted_iota(jnp.int32, (D, D), 0)
    col_i = lax.broadcasted_iota(jnp.int32, (D, D), 1)

    acc = jnp.zeros((N_EX, D), f32)
    for h in range(NHEADS):
        w = watt_ref[h]
        whe = jnp.dot(xe, w, preferred_element_type=f32)
        whk = jnp.dot(xk, w, preferred_element_type=f32)
        u = jnp.dot(whe, asrc_ref[h], preferred_element_type=f32)
        v = jnp.dot(whk, adst_ref[h], preferred_element_type=f32)
        vb = lax.dot_general(ones_col, v, (((1,), (1,)), ((), ())),
                             preferred_element_type=f32)
        e = u + vb
        e = jnp.where(e > 0, e, ALPHA * e)
        e = jnp.where(adj > 0, e, -9e15)
        m = jnp.max(e, axis=1, keepdims=True)
        p = jnp.exp(e - m)
        att = p / jnp.sum(p, axis=1, keepdims=True)
        head = jnp.dot(att, whk, preferred_element_type=f32)
        head = jnp.where(head > 0, head, jnp.exp(head) - 1.0)
        r_h = jnp.where(col_i == 32 * h + row_i // 4, 0.25, 0.0).astype(f32)
        acc = acc + jnp.dot(head, r_h, preferred_element_type=f32)
    e_out[...] = acc


_dense_call = pl.pallas_call(
    _dense_body,
    out_shape=[
        jax.ShapeDtypeStruct((N_EX, D), _F32),
        jax.ShapeDtypeStruct((N_EX, D), _F32),
        jax.ShapeDtypeStruct((N_KC, D), _F32),
    ],
)


_NW = 32
_SLOTS = 4


_CH = 64


def _sc_gather_body(rows_w, te, tr, idxe, idxr, out1, out2,
                    idxe_v, idxr_v, bufs1, bufs2, *sems):
    g1 = sems[:_SLOTS]
    g2 = sems[_SLOTS:2 * _SLOTS]
    w1 = sems[2 * _SLOTS:3 * _SLOTS]
    w2 = sems[3 * _SLOTS:]
    sid = lax.axis_index("s")
    wid = lax.axis_index("c") * 16 + sid
    obase = wid * rows_w

    pltpu.sync_copy(idxe.at[pl.ds(obase, rows_w)], idxe_v)
    pltpu.sync_copy(idxr.at[pl.ds(obase, rows_w)], idxr_v)

    def body(i, carry):
        cps = []
        for k in range(_SLOTS):
            j = i * _SLOTS + k

            @pl.when(i > 0)
            def _():
                pltpu.make_async_copy(
                    bufs1.at[k], out1.at[pl.ds(obase, _CH)], w1[k]).wait()
                pltpu.make_async_copy(
                    bufs2.at[k], out2.at[pl.ds(obase, _CH)], w2[k]).wait()

            sl = pl.ds(j * _CH, _CH)
            cps.append((pltpu.async_copy(te.at[idxe_v.at[sl]], bufs1.at[k],
                                         g1[k]),
                        pltpu.async_copy(tr.at[idxr_v.at[sl]], bufs2.at[k],
                                         g2[k])))
        for k in range(_SLOTS):
            j = i * _SLOTS + k
            cp1, cp2 = cps[k]
            cp1.wait()
            pltpu.async_copy(bufs1.at[k],
                             out1.at[pl.ds(obase + j * _CH, _CH)], w1[k])
            cp2.wait()
            pltpu.async_copy(bufs2.at[k],
                             out2.at[pl.ds(obase + j * _CH, _CH)], w2[k])
        return carry

    lax.fori_loop(0, rows_w // (_CH * _SLOTS), body, 0)
    for k in range(_SLOTS):
        pltpu.make_async_copy(
            bufs1.at[k], out1.at[pl.ds(obase, _CH)], w1[k]).wait()
        pltpu.make_async_copy(
            bufs2.at[k], out2.at[pl.ds(obase, _CH)], w2[k]).wait()


@functools.lru_cache(maxsize=None)
def _make_sc_gather(n_rows):
    rows_w = n_rows // _NW
    mesh = plsc.VectorSubcoreMesh(core_axis_name="c", subcore_axis_name="s")
    return pl.kernel(
        functools.partial(_sc_gather_body, rows_w),
        out_type=[
            jax.ShapeDtypeStruct((n_rows, D), _F32),
            jax.ShapeDtypeStruct((n_rows, 2 * D), _F32),
        ],
        mesh=mesh,
        scratch_types=[
            pltpu.VMEM((rows_w,), jnp.int32),
            pltpu.VMEM((rows_w,), jnp.int32),
            pltpu.VMEM((_SLOTS, _CH, D), _F32),
            pltpu.VMEM((_SLOTS, _CH, 2 * D), _F32),
        ] + [pltpu.SemaphoreType.DMA] * (4 * _SLOTS),
    )


def kernel(exercise_node_embedding, kc_node_mebedding, adj_exercise_kc,
           adj_EE_view, adj_KK_view, exercise_data, exercise_respond_data,
           seqlen, W_att, a_att, W_EE, W_KK):
    b, s = exercise_data.shape
    n_rows = b * s

    asrc = a_att[:, :D]
    adst = a_att[:, D:]

    ex_emb, contrastive_e, contrastive_k = _dense_call(
        exercise_node_embedding, kc_node_mebedding, adj_exercise_kc,
        adj_EE_view, adj_KK_view, W_att, asrc, adst, W_EE, W_KK)

    table_e = jnp.concatenate([jnp.zeros((1, D), _F32), ex_emb], axis=0)
    z = jnp.zeros_like(ex_emb)
    table_r = jnp.concatenate([
        jnp.zeros((1, 2 * D), _F32),
        jnp.concatenate([z, ex_emb], axis=1),
        jnp.concatenate([ex_emb, z], axis=1),
    ], axis=0)

    idxe = exercise_data.T.astype(jnp.int32).reshape(n_rows)
    idxr = exercise_respond_data.T.astype(jnp.int32).reshape(n_rows)

    out1, out2 = _make_sc_gather(n_rows)(table_e, table_r, idxe, idxr)

    return (out2.reshape(s, b, 2 * D),
            out1.reshape(s, b, D),
            ex_emb, contrastive_e, contrastive_k)


# speedup vs baseline: 31.1342x; 1.0378x over previous
import functools

import jax
import jax.numpy as jnp
from jax import lax
from jax.experimental import pallas as pl
from jax.experimental.pallas import tpu as pltpu
from jax.experimental.pallas import tpu_sc as plsc

N_EX = 1000
N_KC = 100
D = 128
NHEADS = 4
ALPHA = 0.2

_F32 = jnp.float32


def _dense_body(xe_ref, xk_ref, adjek_ref, adjee_ref, adjkk_ref, watt_ref,
                asrc_ref, adst_ref, wee_ref, wkk_ref, de_ref, dr_ref,
                e_out, ce_out, ck_out, te_out, tr_out, ie_out, ir_out):
    f32 = _F32
    xe = xe_ref[...]
    xk = xk_ref[...]

    ce_out[...] = jnp.maximum(
        jnp.dot(adjee_ref[...], jnp.dot(xe, wee_ref[...],
                                        preferred_element_type=f32),
                preferred_element_type=f32), 0.0)
    ck_out[...] = jnp.maximum(
        jnp.dot(adjkk_ref[...], jnp.dot(xk, wkk_ref[...],
                                        preferred_element_type=f32),
                preferred_element_type=f32), 0.0)

    adj = adjek_ref[...]
    ones_col = jnp.ones((N_EX, 1), f32)
    row_i = lax.broadcasted_iota(jnp.int32, (D, D), 0)
    col_i = lax.broadcasted_iota(jnp.int32, (D, D), 1)

    acc = jnp.zeros((N_EX, D), f32)
    for h in range(NHEADS):
        w = watt_ref[h]
        whe = jnp.dot(xe, w, preferred_element_type=f32)
        whk = jnp.dot(xk, w, preferred_element_type=f32)
        u = jnp.dot(whe, asrc_ref[h], preferred_element_type=f32)
        v = jnp.dot(whk, adst_ref[h], preferred_element_type=f32)
        vb = lax.dot_general(ones_col, v, (((1,), (1,)), ((), ())),
                             preferred_element_type=f32)
        e = u + vb
        e = jnp.where(e > 0, e, ALPHA * e)
        e = jnp.where(adj > 0, e, -9e15)
        m = jnp.max(e, axis=1, keepdims=True)
        p = jnp.exp(e - m)
        att = p / jnp.sum(p, axis=1, keepdims=True)
        head = jnp.dot(att, whk, preferred_element_type=f32)
        head = jnp.where(head > 0, head, jnp.exp(head) - 1.0)
        r_h = jnp.where(col_i == 32 * h + row_i // 4, 0.25, 0.0).astype(f32)
        acc = acc + jnp.dot(head, r_h, preferred_element_type=f32)
    e_out[...] = acc

    zpad = jnp.zeros((8, D), f32)
    te_out[0:N_EX] = acc
    te_out[N_EX:N_EX + 8] = zpad
    z = jnp.zeros((N_EX, D), f32)
    tr_out[0:N_EX] = jnp.concatenate([z, acc], axis=1)
    tr_out[N_EX:2 * N_EX] = jnp.concatenate([acc, z], axis=1)
    tr_out[2 * N_EX:2 * N_EX + 8] = jnp.concatenate([zpad, zpad], axis=1)

    qe = de_ref[...].T
    ie_out[...] = jnp.where(qe == 0, N_EX, qe - 1)
    qr = dr_ref[...].T
    ir_out[...] = jnp.where(qr == 0, 2 * N_EX, qr - 1)


def _make_dense_call(b, s):
    return pl.pallas_call(
        _dense_body,
        out_shape=[
            jax.ShapeDtypeStruct((N_EX, D), _F32),
            jax.ShapeDtypeStruct((N_EX, D), _F32),
            jax.ShapeDtypeStruct((N_KC, D), _F32),
            jax.ShapeDtypeStruct((N_EX + 8, D), _F32),
            jax.ShapeDtypeStruct((2 * N_EX + 8, 2 * D), _F32),
            jax.ShapeDtypeStruct((s, b), jnp.int32),
            jax.ShapeDtypeStruct((s, b), jnp.int32),
        ],
    )


_NW = 32
_SLOTS = 4


_CH = 64


def _sc_gather_body(rows_w, te, tr, idxe, idxr, out1, out2,
                    idxe_v, idxr_v, bufs1, bufs2, *sems):
    g1 = sems[:_SLOTS]
    g2 = sems[_SLOTS:2 * _SLOTS]
    w1 = sems[2 * _SLOTS:3 * _SLOTS]
    w2 = sems[3 * _SLOTS:]
    sid = lax.axis_index("s")
    wid = lax.axis_index("c") * 16 + sid
    obase = wid * rows_w

    pltpu.sync_copy(idxe.at[pl.ds(obase, rows_w)], idxe_v)
    pltpu.sync_copy(idxr.at[pl.ds(obase, rows_w)], idxr_v)

    def body(i, carry):
        cps = []
        for k in range(_SLOTS):
            j = i * _SLOTS + k

            @pl.when(i > 0)
            def _():
                pltpu.make_async_copy(
                    bufs1.at[k], out1.at[pl.ds(obase, _CH)], w1[k]).wait()
                pltpu.make_async_copy(
                    bufs2.at[k], out2.at[pl.ds(obase, _CH)], w2[k]).wait()

            sl = pl.ds(j * _CH, _CH)
            cps.append((pltpu.async_copy(te.at[idxe_v.at[sl]], bufs1.at[k],
                                         g1[k]),
                        pltpu.async_copy(tr.at[idxr_v.at[sl]], bufs2.at[k],
                                         g2[k])))
        for k in range(_SLOTS):
            j = i * _SLOTS + k
            cp1, cp2 = cps[k]
            cp1.wait()
            pltpu.async_copy(bufs1.at[k],
                             out1.at[pl.ds(obase + j * _CH, _CH)], w1[k])
            cp2.wait()
            pltpu.async_copy(bufs2.at[k],
                             out2.at[pl.ds(obase + j * _CH, _CH)], w2[k])
        return carry

    lax.fori_loop(0, rows_w // (_CH * _SLOTS), body, 0)
    for k in range(_SLOTS):
        pltpu.make_async_copy(
            bufs1.at[k], out1.at[pl.ds(obase, _CH)], w1[k]).wait()
        pltpu.make_async_copy(
            bufs2.at[k], out2.at[pl.ds(obase, _CH)], w2[k]).wait()


@functools.lru_cache(maxsize=None)
def _make_sc_gather(n_rows):
    rows_w = n_rows // _NW
    mesh = plsc.VectorSubcoreMesh(core_axis_name="c", subcore_axis_name="s")
    return pl.kernel(
        functools.partial(_sc_gather_body, rows_w),
        out_type=[
            jax.ShapeDtypeStruct((n_rows, D), _F32),
            jax.ShapeDtypeStruct((n_rows, 2 * D), _F32),
        ],
        mesh=mesh,
        scratch_types=[
            pltpu.VMEM((rows_w,), jnp.int32),
            pltpu.VMEM((rows_w,), jnp.int32),
            pltpu.VMEM((_SLOTS, _CH, D), _F32),
            pltpu.VMEM((_SLOTS, _CH, 2 * D), _F32),
        ] + [pltpu.SemaphoreType.DMA] * (4 * _SLOTS),
    )


def kernel(exercise_node_embedding, kc_node_mebedding, adj_exercise_kc,
           adj_EE_view, adj_KK_view, exercise_data, exercise_respond_data,
           seqlen, W_att, a_att, W_EE, W_KK):
    b, s = exercise_data.shape
    n_rows = b * s

    asrc = a_att[:, :D]
    adst = a_att[:, D:]

    (ex_emb, contrastive_e, contrastive_k, table_e, table_r,
     idxe, idxr) = _make_dense_call(b, s)(
        exercise_node_embedding, kc_node_mebedding, adj_exercise_kc,
        adj_EE_view, adj_KK_view, W_att, asrc, adst, W_EE, W_KK,
        exercise_data.astype(jnp.int32),
        exercise_respond_data.astype(jnp.int32))

    out1, out2 = _make_sc_gather(n_rows)(
        table_e, table_r, idxe.reshape(n_rows), idxr.reshape(n_rows))

    return (out2.reshape(s, b, 2 * D),
            out1.reshape(s, b, D),
            ex_emb, contrastive_e, contrastive_k)
